# baseline (speedup 1.0000x reference)
import jax
import jax.numpy as jnp
from jax import lax
from jax.experimental import pallas as pl
from jax.experimental.pallas import tpu as pltpu
from jax.experimental.pallas import tpu_sc as plsc

_SCALE = 32.0
_MARGIN = 0.2

_B = 1024
_C = 100000
_NC = 2
_NS = 16
_NW = _NC * _NS
_FLAT = _B * _C
_PER_W = _FLAT // _NW
_CPW = _C // _NW
_CH = 10000
_NBUF = 4
_NCH = _PER_W // _CH
_NG = _NCH // _NBUF
_L = 16
_FIX_CH = 128
_NFIX = _B // _FIX_CH


def _sc_body(cos_hbm, lab_hbm, out_hbm, *scratch):
    ins = scratch[0:_NBUF]
    outs = scratch[_NBUF:2 * _NBUF]
    labv, idxv, margv, valv, sem_in, sem_out, sem_g = scratch[2 * _NBUF:]

    wid = lax.axis_index("s") * _NC + lax.axis_index("c")
    base = wid * _PER_W

    def in_cp(i, b):
        return pltpu.make_async_copy(
            cos_hbm.at[pl.ds(base + i * _CH, _CH)], ins[b], sem_in.at[b])

    def out_cp(i, b):
        return pltpu.make_async_copy(
            outs[b], out_hbm.at[pl.ds(base + i * _CH, _CH)], sem_out.at[b])

    for b in range(_NBUF):
        in_cp(b, b).start()

    @pl.loop(0, _NG)
    def _group(g):
        for b in range(_NBUF):
            i = g * _NBUF + b
            in_cp(i, b).wait()

            @pl.when(g > 0)
            def _():
                out_cp(i - _NBUF, b).wait()

            @plsc.parallel_loop(0, _CH, step=_L, unroll=8)
            def _scale(j):
                outs[b][pl.ds(j, _L)] = ins[b][pl.ds(j, _L)] * _SCALE

            out_cp(i, b).start()

            @pl.when(i + _NBUF < _NCH)
            def _():
                in_cp(i + _NBUF, b).start()

    for b in range(_NBUF):
        out_cp(_NCH - _NBUF + b, b).wait()

    pltpu.sync_copy(lab_hbm, labv)
    selmax = jnp.int32(-1)
    for k in range(_NFIX):
        for j in range(_FIX_CH // _L):
            p = k * _FIX_CH + j * _L
            lab = labv[pl.ds(p, _L)]
            row = p + lax.iota(jnp.int32, _L)
            tgt = ((lab >> 3) * 8192 + (row >> 7) * 1024
                   + (lab & 7) * 128 + (row & 127))
            inr = (tgt >= base) & (tgt < base + _PER_W)
            idxv[k, pl.ds(j * _L, _L)] = jnp.where(inr, tgt, base + row)
            margv[pl.ds(p, _L)] = jnp.where(inr, _MARGIN * _SCALE, 0.0)
            selmax = jnp.maximum(selmax, jnp.max(jnp.where(inr, tgt, -1)))
    have = selmax >= 0
    for k in range(_NFIX):
        for j in range(_FIX_CH // _L):
            p = k * _FIX_CH + j * _L
            s = pl.ds(j * _L, _L)
            isdum = margv[pl.ds(p, _L)] == 0.0
            idx = idxv[k, s]
            idxv[k, s] = jnp.where(isdum & have, selmax, idx)
            margv[pl.ds(p, _L)] = jnp.where(
                isdum & have, _MARGIN * _SCALE, margv[pl.ds(p, _L)])
    for k in range(_NFIX):
        pltpu.async_copy(cos_hbm.at[idxv.at[k]], valv, sem_g).wait()
        for j in range(_FIX_CH // _L):
            s = pl.ds(j * _L, _L)
            valv[s] = valv[s] * _SCALE - margv[pl.ds(k * _FIX_CH + j * _L, _L)]
        pltpu.async_copy(valv, out_hbm.at[idxv.at[k]], sem_g).wait()


def _sc_call(cos_flat, lab):
    mesh = plsc.VectorSubcoreMesh(core_axis_name="c", subcore_axis_name="s")
    buf_types = [pltpu.VMEM((_CH,), jnp.float32) for _ in range(2 * _NBUF)]
    return pl.kernel(
        _sc_body,
        out_type=jax.ShapeDtypeStruct((_FLAT,), jnp.float32),
        mesh=mesh,
        compiler_params=pltpu.CompilerParams(needs_layout_passes=False),
        scratch_types=buf_types + [
            pltpu.VMEM((_B,), jnp.int32),
            pltpu.VMEM((_NFIX, _FIX_CH), jnp.int32),
            pltpu.VMEM((_B,), jnp.float32),
            pltpu.VMEM((_FIX_CH,), jnp.float32),
            pltpu.SemaphoreType.DMA((_NBUF,)),
            pltpu.SemaphoreType.DMA((_NBUF,)),
            pltpu.SemaphoreType.DMA,
        ],
    )(cos_flat, lab)


def kernel(cosine, label):
    B, C = cosine.shape
    cos_flat = (cosine.reshape(B // 128, 128, C // 8, 8)
                .transpose(2, 0, 3, 1).reshape(B * C))
    lab = label.astype(jnp.int32)
    out_flat = _sc_call(cos_flat, lab)
    return (out_flat.reshape(C // 8, B // 128, 8, 128)
            .transpose(1, 3, 0, 2).reshape(B, C))


# speedup vs baseline: 3.1655x; 3.1655x over previous
import jax
import jax.numpy as jnp
from jax import lax
from jax.experimental import pallas as pl
from jax.experimental.pallas import tpu as pltpu
from jax.experimental.pallas import tpu_sc as plsc

_SCALE = 32.0
_MARGIN = 0.2

_B = 1024
_C = 100000
_NC = 2
_NS = 16
_NW = _NC * _NS
_RPW = _B // _NW
_L = 16
_CB = 1000


def _tc_scale_body(cos_ref, out_ref):
    out_ref[...] = cos_ref[...] * _SCALE


def _sc_fix_body(lab_hbm, o_hbm, labv, idxv, valv, sem):
    wid = lax.axis_index("s") * _NC + lax.axis_index("c")
    r0 = wid * _RPW
    pltpu.sync_copy(lab_hbm.at[pl.ds(r0, _RPW)], labv)
    for j in range(_RPW // _L):
        row = r0 + j * _L + lax.iota(jnp.int32, _L)
        lab = labv[pl.ds(j * _L, _L)]
        idxv[0, pl.ds(j * _L, _L)] = ((lab >> 3) * 8192 + (row >> 7) * 1024
                                      + (lab & 7) * 128 + (row & 127))
    pltpu.async_copy(o_hbm.at[idxv.at[0]], valv, sem).wait()
    for j in range(_RPW // _L):
        s = pl.ds(j * _L, _L)
        valv[s] = valv[s] - (_MARGIN * _SCALE)
    pltpu.async_copy(valv, o_hbm.at[idxv.at[0]], sem).wait()


def _sc_fixup(lab, o_ref):
    mesh = plsc.VectorSubcoreMesh(core_axis_name="c", subcore_axis_name="s")
    pl.kernel(
        _sc_fix_body,
        mesh=mesh,
        compiler_params=pltpu.CompilerParams(needs_layout_passes=False),
        scratch_types=[
            pltpu.VMEM((_RPW,), jnp.int32),
            pltpu.VMEM((1, _RPW), jnp.int32),
            pltpu.VMEM((_RPW,), jnp.float32),
            pltpu.SemaphoreType.DMA,
        ],
    )(lab, o_ref)


def kernel(cosine, label):
    B, C = cosine.shape
    cos_t = cosine.T
    lab = label.astype(jnp.int32)
    scaled_t = pl.pallas_call(
        _tc_scale_body,
        grid=(C // _CB,),
        in_specs=[pl.BlockSpec((_CB, B), lambda i: (i, 0))],
        out_specs=pl.BlockSpec((_CB, B), lambda i: (i, 0)),
        out_shape=jax.ShapeDtypeStruct((C, B), jnp.float32),
    )(cos_t)
    flat = (scaled_t.reshape(C // 8, 8, B // 128, 128)
            .transpose(0, 2, 1, 3).reshape(C * B))
    o_ref = jax.new_ref(flat)
    _sc_fixup(lab, o_ref)
    out_flat = o_ref[...]
    return (out_flat.reshape(C // 8, B // 128, 8, 128)
            .transpose(1, 3, 0, 2).reshape(B, C))


# speedup vs baseline: 3.1872x; 1.0068x over previous
import jax
import jax.numpy as jnp
from jax import lax
from jax.experimental import pallas as pl
from jax.experimental.pallas import tpu as pltpu
from jax.experimental.pallas import tpu_sc as plsc

_SCALE = 32.0
_MARGIN = 0.2

_B = 1024
_C = 100000
_NC = 2
_NS = 16
_NW = _NC * _NS
_RPW = _B // _NW
_L = 16
_CB = 2000


def _tc_scale_body(cos_ref, out_ref):
    out_ref[...] = cos_ref[...] * _SCALE


def _sc_fix_body(lab_hbm, o_hbm, labv, idxv, valv, sem):
    wid = lax.axis_index("s") * _NC + lax.axis_index("c")
    r0 = wid * _RPW
    pltpu.sync_copy(lab_hbm.at[pl.ds(r0, _RPW)], labv)
    for j in range(_RPW // _L):
        row = r0 + j * _L + lax.iota(jnp.int32, _L)
        lab = labv[pl.ds(j * _L, _L)]
        idxv[0, pl.ds(j * _L, _L)] = ((lab >> 3) * 8192 + (row >> 7) * 1024
                                      + (lab & 7) * 128 + (row & 127))
    pltpu.async_copy(o_hbm.at[idxv.at[0]], valv, sem).wait()
    for j in range(_RPW // _L):
        s = pl.ds(j * _L, _L)
        valv[s] = valv[s] - (_MARGIN * _SCALE)
    pltpu.async_copy(valv, o_hbm.at[idxv.at[0]], sem).wait()


def _sc_fixup(lab, o_ref):
    mesh = plsc.VectorSubcoreMesh(core_axis_name="c", subcore_axis_name="s")
    pl.kernel(
        _sc_fix_body,
        mesh=mesh,
        compiler_params=pltpu.CompilerParams(needs_layout_passes=False),
        scratch_types=[
            pltpu.VMEM((_RPW,), jnp.int32),
            pltpu.VMEM((1, _RPW), jnp.int32),
            pltpu.VMEM((_RPW,), jnp.float32),
            pltpu.SemaphoreType.DMA,
        ],
    )(lab, o_ref)


def kernel(cosine, label):
    B, C = cosine.shape
    cos_t = cosine.T
    lab = label.astype(jnp.int32)
    scaled_t = pl.pallas_call(
        _tc_scale_body,
        grid=(C // _CB,),
        in_specs=[pl.BlockSpec((_CB, B), lambda i: (i, 0))],
        out_specs=pl.BlockSpec((_CB, B), lambda i: (i, 0)),
        out_shape=jax.ShapeDtypeStruct((C, B), jnp.float32),
    )(cos_t)
    flat = (scaled_t.reshape(C // 8, 8, B // 128, 128)
            .transpose(0, 2, 1, 3).reshape(C * B))
    o_ref = jax.new_ref(flat)
    _sc_fixup(lab, o_ref)
    out_flat = o_ref[...]
    return (out_flat.reshape(C // 8, B // 128, 8, 128)
            .transpose(1, 3, 0, 2).reshape(B, C))


# speedup vs baseline: 3.2051x; 1.0056x over previous
import jax
import jax.numpy as jnp
from jax import lax
from jax.experimental import pallas as pl
from jax.experimental.pallas import tpu as pltpu
from jax.experimental.pallas import tpu_sc as plsc

_SCALE = 32.0
_MARGIN = 0.2

_B = 1024
_C = 100000
_NC = 2
_NS = 16
_NW = _NC * _NS
_RPW = _B // _NW
_L = 16
_CB = 4000


def _tc_scale_body(cos_ref, out_ref):
    out_ref[...] = cos_ref[...] * _SCALE


def _sc_fix_body(lab_hbm, o_hbm, labv, idxv, valv, sem):
    wid = lax.axis_index("s") * _NC + lax.axis_index("c")
    r0 = wid * _RPW
    pltpu.sync_copy(lab_hbm.at[pl.ds(r0, _RPW)], labv)
    for j in range(_RPW // _L):
        row = r0 + j * _L + lax.iota(jnp.int32, _L)
        lab = labv[pl.ds(j * _L, _L)]
        idxv[0, pl.ds(j * _L, _L)] = ((lab >> 3) * 8192 + (row >> 7) * 1024
                                      + (lab & 7) * 128 + (row & 127))
    pltpu.async_copy(o_hbm.at[idxv.at[0]], valv, sem).wait()
    for j in range(_RPW // _L):
        s = pl.ds(j * _L, _L)
        valv[s] = valv[s] - (_MARGIN * _SCALE)
    pltpu.async_copy(valv, o_hbm.at[idxv.at[0]], sem).wait()


def _sc_fixup(lab, o_ref):
    mesh = plsc.VectorSubcoreMesh(core_axis_name="c", subcore_axis_name="s")
    pl.kernel(
        _sc_fix_body,
        mesh=mesh,
        compiler_params=pltpu.CompilerParams(needs_layout_passes=False),
        scratch_types=[
            pltpu.VMEM((_RPW,), jnp.int32),
            pltpu.VMEM((1, _RPW), jnp.int32),
            pltpu.VMEM((_RPW,), jnp.float32),
            pltpu.SemaphoreType.DMA,
        ],
    )(lab, o_ref)


def kernel(cosine, label):
    B, C = cosine.shape
    cos_t = cosine.T
    lab = label.astype(jnp.int32)
    scaled_t = pl.pallas_call(
        _tc_scale_body,
        grid=(C // _CB,),
        in_specs=[pl.BlockSpec((_CB, B), lambda i: (i, 0))],
        out_specs=pl.BlockSpec((_CB, B), lambda i: (i, 0)),
        out_shape=jax.ShapeDtypeStruct((C, B), jnp.float32),
        compiler_params=pltpu.CompilerParams(
            vmem_limit_bytes=128 * 1024 * 1024),
    )(cos_t)
    flat = (scaled_t.reshape(C // 8, 8, B // 128, 128)
            .transpose(0, 2, 1, 3).reshape(C * B))
    o_ref = jax.new_ref(flat)
    _sc_fixup(lab, o_ref)
    out_flat = o_ref[...]
    return (out_flat.reshape(C // 8, B // 128, 8, 128)
            .transpose(1, 3, 0, 2).reshape(B, C))
